# baseline (speedup 1.0000x reference)
import functools

import jax
import jax.numpy as jnp
from jax import lax
from jax.experimental import pallas as pl
from jax.experimental.pallas import tpu as pltpu
from jax.experimental.pallas import tpu_sc as plsc

N = 10000
K = 16
CIN = 128
COUT = 32
M = 4

NW = 32
N_PAD = 10240
PER_W = N_PAD // NW
C = 8
N_CHUNKS = PER_W // C
N_PAIRS = N_CHUNKS // 2


def _mm_body(x_ref, w_ref, y_ref):
    y_ref[...] = jnp.dot(x_ref[...], w_ref[...],
                         preferred_element_type=jnp.float32)


def _tc_matmul(x2, wcat):
    blk = 2048
    return pl.pallas_call(
        _mm_body,
        grid=(N_PAD // blk,),
        in_specs=[pl.BlockSpec((blk, CIN), lambda i: (i, 0)),
                  pl.BlockSpec((CIN, 256), lambda i: (0, 0)),],
        out_specs=pl.BlockSpec((blk, 256), lambda i: (i, 0)),
        out_shape=jax.ShapeDtypeStruct((N_PAD, 256), jnp.float32),
    )(x2, wcat)


def _sc_body(wx_hbm, uxf_hbm, adjf_hbm, c_hbm, b_hbm, out_hbm,
             idx_a, idx_b, adj_all, wrows_a, wrows_b, uxf_v, out_all,
             cvec, bvec, sem_a, sem_b):
    wid = lax.axis_index("c") * 16 + lax.axis_index("s")
    base_w = wid * PER_W
    pltpu.sync_copy(c_hbm, cvec)
    pltpu.sync_copy(b_hbm, bvec)
    pltpu.sync_copy(adjf_hbm.at[pl.ds(base_w * K, PER_W * K)], adj_all)
    pltpu.sync_copy(uxf_hbm, uxf_v)
    cv = cvec[...]
    cs = [cv[m] for m in range(M)]
    b_lo = bvec[pl.ds(0, 16)]
    b_hi = bvec[pl.ds(16, 16)]

    def build_idx(idx_ref, ci):
        for cc in range(C):
            a = adj_all[pl.ds((ci * C + cc) * K, K)]
            idx_ref[pl.ds(cc * K, K)] = jnp.maximum(a - 1, 0)

    def compute_chunk(wrows, ci):
        def node_body(cc, _):
            loc = ci * C + cc
            a = adj_all[pl.ds(loc * K, K)]
            valid = a > 0
            cnt = jnp.zeros((16,), jnp.float32) + jnp.sum(
                jnp.where(valid, 1.0, 0.0))
            invc = jnp.where(cnt > 0.0, 1.0 / cnt, 0.0)
            idx0 = jnp.maximum(a - 1, 0)
            base4 = idx0 * M
            own = (base_w + loc) * M
            ps = []
            for m in range(M):
                uxg = plsc.load_gather(uxf_v, [base4 + m])
                uo = plsc.load_gather(
                    uxf_v, [jnp.full((16,), m, jnp.int32) + own])
                ps.append((uo + cs[m]) - uxg)
            pmax = jnp.maximum(jnp.maximum(ps[0], ps[1]),
                               jnp.maximum(ps[2], ps[3]))
            es = [jnp.exp(p - pmax) for p in ps]
            ssum = (es[0] + es[1]) + (es[2] + es[3])
            scale = invc / ssum
            wms = [jnp.where(valid, e * scale, 0.0) for e in es]
            acc_lo = b_lo
            acc_hi = b_hi
            for k in range(K):
                j = cc * K + k
                for m in range(M):
                    w = wms[m][k]
                    acc_lo = acc_lo + w * wrows[j, pl.ds(32 * m, 16)]
                    acc_hi = acc_hi + w * wrows[j, pl.ds(32 * m + 16, 16)]
            out_all[pl.ds(loc * COUT, 16)] = acc_lo
            out_all[pl.ds(loc * COUT + 16, 16)] = acc_hi
            return 0

        lax.fori_loop(0, C, node_body, 0)

    build_idx(idx_a, 0)
    pltpu.async_copy(wx_hbm.at[idx_a], wrows_a, sem_a)

    def pair_body(i, _):
        build_idx(idx_b, 2 * i + 1)
        cp_b = pltpu.async_copy(wx_hbm.at[idx_b], wrows_b, sem_b)
        pltpu.make_async_copy(wx_hbm.at[idx_a], wrows_a, sem_a).wait()
        compute_chunk(wrows_a, 2 * i)

        @pl.when(i < N_PAIRS - 1)
        def _():
            build_idx(idx_a, 2 * i + 2)
            pltpu.async_copy(wx_hbm.at[idx_a], wrows_a, sem_a)

        cp_b.wait()
        compute_chunk(wrows_b, 2 * i + 1)
        return 0

    lax.fori_loop(0, N_PAIRS, pair_body, 0)
    pltpu.sync_copy(out_all, out_hbm.at[pl.ds(base_w * COUT, PER_W * COUT)])


_sc_kernel = functools.partial(
    pl.kernel,
    mesh=plsc.VectorSubcoreMesh(core_axis_name="c", subcore_axis_name="s"),
    compiler_params=pltpu.CompilerParams(needs_layout_passes=False),
    out_type=jax.ShapeDtypeStruct((N_PAD * COUT,), jnp.float32),
    scratch_types=[
        pltpu.VMEM((C * K,), jnp.int32),
        pltpu.VMEM((C * K,), jnp.int32),
        pltpu.VMEM((PER_W * K,), jnp.int32),
        pltpu.VMEM((C * K, CIN), jnp.float32),
        pltpu.VMEM((C * K, CIN), jnp.float32),
        pltpu.VMEM((N_PAD * M,), jnp.float32),
        pltpu.VMEM((PER_W * COUT,), jnp.float32),
        pltpu.VMEM((16,), jnp.float32),
        pltpu.VMEM((COUT,), jnp.float32),
        pltpu.SemaphoreType.DMA,
        pltpu.SemaphoreType.DMA,
    ],
)(_sc_body)


def kernel(x, adj, W, b, u, c):
    x2 = x[0]
    x2p = jnp.pad(x2, ((0, N_PAD - N), (0, 0)))
    Wr = W.reshape(M * COUT, CIN)
    wcat = jnp.concatenate(
        [Wr.T, u.T, jnp.zeros((CIN, 256 - M * COUT - M), jnp.float32)],
        axis=1)
    y = _tc_matmul(x2p, wcat)
    wx = y[:, :M * COUT]
    uxf = y[:, M * COUT:M * COUT + M].reshape(-1)
    adjf = jnp.pad(adj, ((0, N_PAD - N), (0, 0))).reshape(-1)
    c_pad = jnp.pad(c, (0, 16 - M))
    out = _sc_kernel(wx, uxf, adjf, c_pad, b)
    return out[:N * COUT].reshape(1, N, COUT)


# speedup vs baseline: 1.0529x; 1.0529x over previous
import functools

import jax
import jax.numpy as jnp
from jax import lax
from jax.experimental import pallas as pl
from jax.experimental.pallas import tpu as pltpu
from jax.experimental.pallas import tpu_sc as plsc

N = 10000
K = 16
CIN = 128
COUT = 32
M = 4

NW = 32
N_PAD = 10240
C = 8
PER_W0 = 496
PER_W1 = 144
PER_W_MAX = PER_W0
CORE1_BASE = PER_W0 * 16


def _mm_body(x_ref, w_ref, y_ref):
    y_ref[...] = jnp.dot(x_ref[...], w_ref[...],
                         preferred_element_type=jnp.float32)


def _tc_matmul(x2, wcat):
    blk = 2048
    return pl.pallas_call(
        _mm_body,
        grid=(N_PAD // blk,),
        in_specs=[pl.BlockSpec((blk, CIN), lambda i: (i, 0)),
                  pl.BlockSpec((CIN, 256), lambda i: (0, 0)),],
        out_specs=pl.BlockSpec((blk, 256), lambda i: (i, 0)),
        out_shape=jax.ShapeDtypeStruct((N_PAD, 256), jnp.float32),
    )(x2, wcat)


def _sc_body(wx_hbm, uxf_hbm, adjf_hbm, c_hbm, b_hbm, out_hbm,
             idx_a, idx_b, adj_all, wrows_a, wrows_b, uxf_v, out_all,
             cvec, bvec, sem_a, sem_b):
    cid = lax.axis_index("c")
    sid = lax.axis_index("s")
    pltpu.sync_copy(c_hbm, cvec)
    pltpu.sync_copy(b_hbm, bvec)
    pltpu.sync_copy(uxf_hbm, uxf_v)
    cv = cvec[...]
    cs = [cv[m] for m in range(M)]
    b_lo = bvec[pl.ds(0, 16)]
    b_hi = bvec[pl.ds(16, 16)]

    def worker(base_w, per_w):
        n_pairs = per_w // C // 2
        pltpu.sync_copy(adjf_hbm.at[pl.ds(base_w * K, per_w * K)],
                        adj_all.at[pl.ds(0, per_w * K)])

        def build_idx(idx_ref, ci):
            for cc in range(C):
                a = adj_all[pl.ds((ci * C + cc) * K, K)]
                idx_ref[pl.ds(cc * K, K)] = jnp.maximum(a - 1, 0)

        def compute_chunk(wrows, ci):
            def node_body(cc, _):
                loc = ci * C + cc
                a = adj_all[pl.ds(loc * K, K)]
                valid = a > 0
                cnt = jnp.zeros((16,), jnp.float32) + jnp.sum(
                    jnp.where(valid, 1.0, 0.0))
                invc = jnp.where(cnt > 0.0, 1.0 / cnt, 0.0)
                idx0 = jnp.maximum(a - 1, 0)
                base4 = idx0 * M
                own = (base_w + loc) * M
                ps = []
                for m in range(M):
                    uxg = plsc.load_gather(uxf_v, [base4 + m])
                    uo = plsc.load_gather(
                        uxf_v, [jnp.full((16,), m, jnp.int32) + own])
                    ps.append((uo + cs[m]) - uxg)
                pmax = jnp.maximum(jnp.maximum(ps[0], ps[1]),
                                   jnp.maximum(ps[2], ps[3]))
                es = [jnp.exp(p - pmax) for p in ps]
                ssum = (es[0] + es[1]) + (es[2] + es[3])
                scale = invc / ssum
                wms = [jnp.where(valid, e * scale, 0.0) for e in es]
                acc_lo = b_lo
                acc_hi = b_hi
                for k in range(K):
                    j = cc * K + k
                    for m in range(M):
                        w = wms[m][k]
                        acc_lo = acc_lo + w * wrows[j, pl.ds(32 * m, 16)]
                        acc_hi = acc_hi + w * wrows[j, pl.ds(32 * m + 16, 16)]
                out_all[pl.ds(loc * COUT, 16)] = acc_lo
                out_all[pl.ds(loc * COUT + 16, 16)] = acc_hi
                return 0

            lax.fori_loop(0, C, node_body, 0)

        def pair_body(i, _):
            build_idx(idx_b, 2 * i + 1)
            cp_b = pltpu.async_copy(wx_hbm.at[idx_b], wrows_b, sem_b)
            pltpu.make_async_copy(wx_hbm.at[idx_a], wrows_a, sem_a).wait()
            compute_chunk(wrows_a, 2 * i)

            @pl.when(i < n_pairs - 1)
            def _():
                build_idx(idx_a, 2 * i + 2)
                pltpu.async_copy(wx_hbm.at[idx_a], wrows_a, sem_a)

            cp_b.wait()
            compute_chunk(wrows_b, 2 * i + 1)
            return 0

        build_idx(idx_a, 0)
        pltpu.async_copy(wx_hbm.at[idx_a], wrows_a, sem_a)
        lax.fori_loop(0, n_pairs, pair_body, 0)
        pltpu.sync_copy(out_all.at[pl.ds(0, per_w * COUT)],
                        out_hbm.at[pl.ds(base_w * COUT, per_w * COUT)])

    @pl.when(cid == 0)
    def _():
        worker(sid * PER_W0, PER_W0)

    @pl.when(cid == 1)
    def _():
        worker(CORE1_BASE + sid * PER_W1, PER_W1)


_sc_kernel = functools.partial(
    pl.kernel,
    mesh=plsc.VectorSubcoreMesh(core_axis_name="c", subcore_axis_name="s"),
    compiler_params=pltpu.CompilerParams(needs_layout_passes=False),
    out_type=jax.ShapeDtypeStruct((N_PAD * COUT,), jnp.float32),
    scratch_types=[
        pltpu.VMEM((C * K,), jnp.int32),
        pltpu.VMEM((C * K,), jnp.int32),
        pltpu.VMEM((PER_W_MAX * K,), jnp.int32),
        pltpu.VMEM((C * K, CIN), jnp.float32),
        pltpu.VMEM((C * K, CIN), jnp.float32),
        pltpu.VMEM((N_PAD * M,), jnp.float32),
        pltpu.VMEM((PER_W_MAX * COUT,), jnp.float32),
        pltpu.VMEM((16,), jnp.float32),
        pltpu.VMEM((COUT,), jnp.float32),
        pltpu.SemaphoreType.DMA,
        pltpu.SemaphoreType.DMA,
    ],
)(_sc_body)


def kernel(x, adj, W, b, u, c):
    x2 = x[0]
    x2p = jnp.pad(x2, ((0, N_PAD - N), (0, 0)))
    Wr = W.reshape(M * COUT, CIN)
    wcat = jnp.concatenate(
        [Wr.T, u.T, jnp.zeros((CIN, 256 - M * COUT - M), jnp.float32)],
        axis=1)
    y = _tc_matmul(x2p, wcat)
    wx = y[:, :M * COUT]
    uxf = y[:, M * COUT:M * COUT + M].reshape(-1)
    adjf = jnp.pad(adj, ((0, N_PAD - N), (0, 0))).reshape(-1)
    c_pad = jnp.pad(c, (0, 16 - M))
    out = _sc_kernel(wx, uxf, adjf, c_pad, b)
    return out[:N * COUT].reshape(1, N, COUT)
